# baseline (speedup 1.0000x reference)
import functools

import jax
import jax.numpy as jnp
from jax import lax
from jax.experimental import pallas as pl
from jax.experimental.pallas import tpu as pltpu
from jax.experimental.pallas import tpu_sc as plsc

IMG_SIZE = 512
B, N, D = 2, 5000, 256
HID, NCLS = 4096, 81
NMS_KEEP = 100
IOU_THRESH = 0.5

NPAD = 5120
ROW_T = 512
NSUB = 16
LANES = 16
CHUNK = NPAD // NSUB
KPAD = 128


def _stage1_body(rois_ref, p0_ref, p1_ref, p2_ref, p3_ref, rowid_ref,
                 w1_ref, b1_ref, wcr_ref, bcr_ref,
                 s_o, x_o, y_o, w_o, h_o, x1_o, y1_o, ar_o):
    x = jnp.dot(rois_ref[0], w1_ref[...],
                preferred_element_type=jnp.float32) + b1_ref[...]
    y = jnp.dot(x, wcr_ref[...], preferred_element_type=jnp.float32) + bcr_ref[...]
    clss = y[:, :NCLS]
    reg = y[:, NCLS:NCLS + 4]
    m = jnp.max(clss, axis=1)
    ssum = jnp.sum(jnp.exp(clss - m[:, None]), axis=1)
    score = 1.0 / ssum
    valid = (clss[:, 0] < m) & (score >= 0.01) & (rowid_ref[0] < N)
    score = jnp.where(valid, score, 0.0)

    p0 = p0_ref[...]
    p1 = p1_ref[...]
    p2 = p2_ref[...]
    p3 = p3_ref[...]
    px = p0 + p2 * reg[:, 0]
    py = p1 + p3 * reg[:, 1]
    pw = p2 * jnp.exp(reg[:, 2])
    ph = p3 * jnp.exp(reg[:, 3])
    bx = jnp.clip(px, 0.0, IMG_SIZE - 1.0)
    by = jnp.clip(py, 0.0, IMG_SIZE - 1.0)
    bw = jnp.clip(pw, 1.0, float(IMG_SIZE))
    bh = jnp.clip(ph, 1.0, float(IMG_SIZE))
    x1 = bx + bw - 1.0
    y1 = by + bh - 1.0
    area = (x1 - bx + 1.0) * (y1 - by + 1.0)

    s_o[...] = score
    x_o[...] = bx
    y_o[...] = by
    w_o[...] = bw
    h_o[...] = bh
    x1_o[...] = x1
    y1_o[...] = y1
    ar_o[...] = area


def _stage1(rois_p, p0, p1, p2, p3, rowid, W1, b1r, Wcr, bcr):
    nt = NPAD // ROW_T
    flat = pl.BlockSpec((ROW_T,), lambda b, t: (b * nt + t,))
    return pl.pallas_call(
        _stage1_body,
        grid=(B, nt),
        in_specs=[
            pl.BlockSpec((1, ROW_T, D), lambda b, t: (b, t, 0)),
            flat, flat, flat, flat,
            pl.BlockSpec((1, ROW_T), lambda b, t: (0, t)),
            pl.BlockSpec((D, HID), lambda b, t: (0, 0)),
            pl.BlockSpec((1, HID), lambda b, t: (0, 0)),
            pl.BlockSpec((HID, NCLS + 4), lambda b, t: (0, 0)),
            pl.BlockSpec((1, NCLS + 4), lambda b, t: (0, 0)),
        ],
        out_specs=[flat] * 8,
        out_shape=[jax.ShapeDtypeStruct((B * NPAD,), jnp.float32)] * 8,
    )(rois_p, p0, p1, p2, p3, rowid, W1, b1r, Wcr, bcr)


def _nms_body(s_hbm, x_hbm, y_hbm, w_hbm, h_hbm, x1_hbm, y1_hbm, ar_hbm,
              out_s_hbm, out_b_hbm,
              s_ref, ox0, oy0, ox1, oy1, oar,
              fx, fy, fw, fh, fx1, fy1, far,
              ks, kx, ky, kw, kh,
              stg, allv, shx):
    img = lax.axis_index("c")
    sid = lax.axis_index("s")
    base = sid * CHUNK
    ib = img * NPAD

    pltpu.sync_copy(s_hbm.at[pl.ds(ib + base, CHUNK)], s_ref)
    pltpu.sync_copy(x_hbm.at[pl.ds(ib + base, CHUNK)], ox0)
    pltpu.sync_copy(y_hbm.at[pl.ds(ib + base, CHUNK)], oy0)
    pltpu.sync_copy(x1_hbm.at[pl.ds(ib + base, CHUNK)], ox1)
    pltpu.sync_copy(y1_hbm.at[pl.ds(ib + base, CHUNK)], oy1)
    pltpu.sync_copy(ar_hbm.at[pl.ds(ib + base, CHUNK)], oar)
    pltpu.sync_copy(x_hbm.at[pl.ds(ib, NPAD)], fx)
    pltpu.sync_copy(y_hbm.at[pl.ds(ib, NPAD)], fy)
    pltpu.sync_copy(w_hbm.at[pl.ds(ib, NPAD)], fw)
    pltpu.sync_copy(h_hbm.at[pl.ds(ib, NPAD)], fh)
    pltpu.sync_copy(x1_hbm.at[pl.ds(ib, NPAD)], fx1)
    pltpu.sync_copy(y1_hbm.at[pl.ds(ib, NPAD)], fy1)
    pltpu.sync_copy(ar_hbm.at[pl.ds(ib, NPAD)], far)

    lanes = lax.iota(jnp.int32, LANES)
    lane0 = lanes == 0
    neg1 = jnp.full((LANES,), -1.0, jnp.float32)

    def _rot(v, sh):
        idx = (lanes + sh) & (LANES - 1)
        return lax.gather(
            v, idx[:, None],
            dimension_numbers=lax.GatherDimensionNumbers(
                offset_dims=(), collapsed_slice_dims=(0,),
                start_index_map=(0,)),
            slice_sizes=(1,),
            mode=lax.GatherScatterMode.PROMISE_IN_BOUNDS)

    def _lane_argmax(m, ix):
        for sh in (1, 2, 4, 8):
            m2 = _rot(m, sh)
            ix2 = _rot(ix, sh)
            better = (m2 > m) | ((m2 == m) & (ix2 < ix))
            m = jnp.where(better, m2, m)
            ix = jnp.where(better, ix2, ix)
        return m, ix

    m0 = s_ref[pl.ds(0, LANES)]
    ix0 = lanes + base
    for j in range(1, CHUNK // LANES):
        v = s_ref[pl.ds(j * LANES, LANES)]
        take = v > m0
        m0 = jnp.where(take, v, m0)
        ix0 = jnp.where(take, lanes + (base + j * LANES), ix0)
    m0, ix0 = _lane_argmax(m0, ix0)

    def step(k, carry):
        m, ix = carry

        p = k & 1
        stg[pl.ds(0, LANES)] = m
        stg[pl.ds(LANES, LANES)] = plsc.bitcast(ix, jnp.float32)
        pltpu.sync_copy(stg, shx.at[p, sid, pl.ds(0, 2 * LANES)])
        plsc.subcore_barrier()
        pltpu.sync_copy(shx.at[p], allv)
        wm = allv[0, pl.ds(0, LANES)]
        wi = plsc.bitcast(allv[0, pl.ds(LANES, LANES)], jnp.int32)
        for t in range(1, NSUB):
            vm = allv[t, pl.ds(0, LANES)]
            vi = plsc.bitcast(allv[t, pl.ds(LANES, LANES)], jnp.int32)
            better = (vm > wm) | ((vm == wm) & (vi < wi))
            wm = jnp.where(better, vm, wm)
            wi = jnp.where(better, vi, wi)

        wx = plsc.load_gather(fx, [wi])
        wy = plsc.load_gather(fy, [wi])
        ww = plsc.load_gather(fw, [wi])
        wh = plsc.load_gather(fh, [wi])
        wx1 = plsc.load_gather(fx1, [wi])
        wy1 = plsc.load_gather(fy1, [wi])
        wa = plsc.load_gather(far, [wi])

        ok = wm > 0.0
        kidx = jnp.full((LANES,), k, jnp.int32)
        zero = jnp.zeros((LANES,), jnp.float32)
        plsc.store_scatter(ks, [kidx], jnp.where(ok, wm, zero), mask=lane0)
        plsc.store_scatter(kx, [kidx], jnp.where(ok, wx, zero), mask=lane0)
        plsc.store_scatter(ky, [kidx], jnp.where(ok, wy, zero), mask=lane0)
        plsc.store_scatter(kw, [kidx], jnp.where(ok, ww, zero), mask=lane0)
        plsc.store_scatter(kh, [kidx], jnp.where(ok, wh, zero), mask=lane0)

        nm = neg1
        nix = lanes + base
        for j in range(CHUNK // LANES):
            sl = pl.ds(j * LANES, LANES)
            x0 = ox0[sl]
            y0 = oy0[sl]
            x1 = ox1[sl]
            y1 = oy1[sl]
            iw = jnp.maximum(jnp.minimum(wx1, x1) - jnp.maximum(wx, x0) + 1.0, 0.0)
            ih = jnp.maximum(jnp.minimum(wy1, y1) - jnp.maximum(wy, y0) + 1.0, 0.0)
            inter = iw * ih
            iou = inter / (wa + oar[sl] - inter + 1e-9)
            s = jnp.where(iou >= IOU_THRESH, neg1, s_ref[sl])
            s_ref[sl] = s
            take = s > nm
            nm = jnp.where(take, s, nm)
            nix = jnp.where(take, lanes + (base + j * LANES), nix)
        return _lane_argmax(nm, nix)

    lax.fori_loop(0, NMS_KEEP, step, (m0, ix0))

    @pl.when(sid == 0)
    def _():
        for j in range(KPAD // LANES):
            sl = pl.ds(j * LANES, LANES)
            kx1 = kx[sl] + kw[sl] - 1.0
            ky1 = ky[sl] + kh[sl] - 1.0
            kw[sl] = kx1
            kh[sl] = ky1
        ob = img * (4 * KPAD)
        pltpu.sync_copy(ks, out_s_hbm.at[pl.ds(img * KPAD, KPAD)])
        pltpu.sync_copy(kx, out_b_hbm.at[pl.ds(ob + 0 * KPAD, KPAD)])
        pltpu.sync_copy(ky, out_b_hbm.at[pl.ds(ob + 1 * KPAD, KPAD)])
        pltpu.sync_copy(kw, out_b_hbm.at[pl.ds(ob + 2 * KPAD, KPAD)])
        pltpu.sync_copy(kh, out_b_hbm.at[pl.ds(ob + 3 * KPAD, KPAD)])


@functools.cache
def _make_nms_sc():
    return functools.partial(
        pl.kernel,
        out_type=[
            jax.ShapeDtypeStruct((B * KPAD,), jnp.float32),
            jax.ShapeDtypeStruct((B * 4 * KPAD,), jnp.float32),
        ],
        mesh=plsc.VectorSubcoreMesh(core_axis_name="c", subcore_axis_name="s"),
        compiler_params=pltpu.CompilerParams(needs_layout_passes=False),
        scratch_types=[
        pltpu.VMEM((CHUNK,), jnp.float32),
        pltpu.VMEM((CHUNK,), jnp.float32),
        pltpu.VMEM((CHUNK,), jnp.float32),
        pltpu.VMEM((CHUNK,), jnp.float32),
        pltpu.VMEM((CHUNK,), jnp.float32),
        pltpu.VMEM((CHUNK,), jnp.float32),
        pltpu.VMEM((NPAD,), jnp.float32),
        pltpu.VMEM((NPAD,), jnp.float32),
        pltpu.VMEM((NPAD,), jnp.float32),
        pltpu.VMEM((NPAD,), jnp.float32),
        pltpu.VMEM((NPAD,), jnp.float32),
        pltpu.VMEM((NPAD,), jnp.float32),
        pltpu.VMEM((NPAD,), jnp.float32),
        pltpu.VMEM((KPAD,), jnp.float32),
        pltpu.VMEM((KPAD,), jnp.float32),
        pltpu.VMEM((KPAD,), jnp.float32),
        pltpu.VMEM((KPAD,), jnp.float32),
        pltpu.VMEM((KPAD,), jnp.float32),
        pltpu.VMEM((2 * LANES,), jnp.float32),
        pltpu.VMEM((NSUB, 128), jnp.float32),
            pltpu.VMEM_SHARED((2, NSUB, 128), jnp.float32),
        ],
    )(_nms_body)


def kernel(rois, proposals, W1, b1, Wc, bc, Wr, br):
    rois = rois.reshape(B, N, D)
    pad = NPAD - N
    rois_p = jnp.pad(rois, ((0, 0), (0, pad), (0, 0)))
    prop_p = jnp.pad(proposals, ((0, 0), (0, pad), (0, 0)))
    pcols = [prop_p[:, :, j].reshape(-1) for j in range(4)]
    rowid = jnp.arange(NPAD, dtype=jnp.int32)[None, :]
    Wcr = jnp.concatenate([Wc, Wr], axis=1)
    bcr = jnp.concatenate([bc, br], axis=0)[None, :]
    packed = _stage1(rois_p, *pcols, rowid, W1, b1[None, :], Wcr, bcr)
    out_s, out_b = _make_nms_sc()(*packed)
    clss_out = out_s.reshape(B, KPAD)[:, :NMS_KEEP]
    out_b = out_b.reshape(B, 4, KPAD)
    bboxes_out = jnp.transpose(out_b, (0, 2, 1))[:, :NMS_KEEP, :]
    return (clss_out, bboxes_out)


# speedup vs baseline: 1.1417x; 1.1417x over previous
import functools

import jax
import jax.numpy as jnp
from jax import lax
from jax.experimental import pallas as pl
from jax.experimental.pallas import tpu as pltpu
from jax.experimental.pallas import tpu_sc as plsc

IMG_SIZE = 512
B, N, D = 2, 5000, 256
HID, NCLS = 4096, 81
NMS_KEEP = 100
IOU_THRESH = 0.5

NPAD = 5120
ROW_T = 1024
NSUB = 16
LANES = 16
CHUNK = NPAD // NSUB
KPAD = 128


def _stage1_body(rois_ref, p0_ref, p1_ref, p2_ref, p3_ref, rowid_ref,
                 w1_ref, b1_ref, wcr_ref, bcr_ref,
                 s_o, x_o, y_o, w_o, h_o, x1_o, y1_o, ar_o):
    x = jnp.dot(rois_ref[0], w1_ref[...],
                preferred_element_type=jnp.float32) + b1_ref[...]
    y = jnp.dot(x, wcr_ref[...], preferred_element_type=jnp.float32) + bcr_ref[...]
    clss = y[:, :NCLS]
    m2 = jnp.max(clss, axis=1, keepdims=True)
    ssum2 = jnp.sum(jnp.exp(clss - m2), axis=1, keepdims=True)
    z = jnp.concatenate(
        [m2, ssum2, clss[:, 0:1], y[:, NCLS:NCLS + 1], y[:, NCLS + 1:NCLS + 2],
         y[:, NCLS + 2:NCLS + 3], y[:, NCLS + 3:NCLS + 4], m2], axis=1)
    zT = z.T
    m = zT[0, :]
    ssum = zT[1, :]
    c0 = zT[2, :]
    reg0 = zT[3, :]
    reg1 = zT[4, :]
    reg2 = zT[5, :]
    reg3 = zT[6, :]
    score = 1.0 / ssum
    valid = (c0 < m) & (score >= 0.01) & (rowid_ref[0] < N)
    score = jnp.where(valid, score, 0.0)

    p0 = p0_ref[...]
    p1 = p1_ref[...]
    p2 = p2_ref[...]
    p3 = p3_ref[...]
    px = p0 + p2 * reg0
    py = p1 + p3 * reg1
    pw = p2 * jnp.exp(reg2)
    ph = p3 * jnp.exp(reg3)
    bx = jnp.clip(px, 0.0, IMG_SIZE - 1.0)
    by = jnp.clip(py, 0.0, IMG_SIZE - 1.0)
    bw = jnp.clip(pw, 1.0, float(IMG_SIZE))
    bh = jnp.clip(ph, 1.0, float(IMG_SIZE))
    x1 = bx + bw - 1.0
    y1 = by + bh - 1.0
    area = (x1 - bx + 1.0) * (y1 - by + 1.0)

    s_o[...] = score
    x_o[...] = bx
    y_o[...] = by
    w_o[...] = bw
    h_o[...] = bh
    x1_o[...] = x1
    y1_o[...] = y1
    ar_o[...] = area


def _stage1(rois_p, p0, p1, p2, p3, rowid, W1, b1r, Wcr, bcr):
    nt = NPAD // ROW_T
    flat = pl.BlockSpec((ROW_T,), lambda b, t: (b * nt + t,))
    return pl.pallas_call(
        _stage1_body,
        grid=(B, nt),
        in_specs=[
            pl.BlockSpec((1, ROW_T, D), lambda b, t: (b, t, 0)),
            flat, flat, flat, flat,
            pl.BlockSpec((1, ROW_T), lambda b, t: (0, t)),
            pl.BlockSpec((D, HID), lambda b, t: (0, 0)),
            pl.BlockSpec((1, HID), lambda b, t: (0, 0)),
            pl.BlockSpec((HID, NCLS + 4), lambda b, t: (0, 0)),
            pl.BlockSpec((1, NCLS + 4), lambda b, t: (0, 0)),
        ],
        out_specs=[flat] * 8,
        out_shape=[jax.ShapeDtypeStruct((B * NPAD,), jnp.float32)] * 8,
    )(rois_p, p0, p1, p2, p3, rowid, W1, b1r, Wcr, bcr)


def _nms_body(s_hbm, x_hbm, y_hbm, w_hbm, h_hbm, x1_hbm, y1_hbm, ar_hbm,
              out_s_hbm, out_b_hbm,
              s_ref, ox0, oy0, ox1, oy1, oar,
              fx, fy, fw, fh, fx1, fy1, far,
              ks, kx, ky, kw, kh,
              stg, allv, shx):
    img = lax.axis_index("c")
    sid = lax.axis_index("s")
    base = sid * CHUNK
    ib = img * NPAD

    pltpu.sync_copy(s_hbm.at[pl.ds(ib + base, CHUNK)], s_ref)
    pltpu.sync_copy(x_hbm.at[pl.ds(ib + base, CHUNK)], ox0)
    pltpu.sync_copy(y_hbm.at[pl.ds(ib + base, CHUNK)], oy0)
    pltpu.sync_copy(x1_hbm.at[pl.ds(ib + base, CHUNK)], ox1)
    pltpu.sync_copy(y1_hbm.at[pl.ds(ib + base, CHUNK)], oy1)
    pltpu.sync_copy(ar_hbm.at[pl.ds(ib + base, CHUNK)], oar)
    pltpu.sync_copy(x_hbm.at[pl.ds(ib, NPAD)], fx)
    pltpu.sync_copy(y_hbm.at[pl.ds(ib, NPAD)], fy)
    pltpu.sync_copy(w_hbm.at[pl.ds(ib, NPAD)], fw)
    pltpu.sync_copy(h_hbm.at[pl.ds(ib, NPAD)], fh)
    pltpu.sync_copy(x1_hbm.at[pl.ds(ib, NPAD)], fx1)
    pltpu.sync_copy(y1_hbm.at[pl.ds(ib, NPAD)], fy1)
    pltpu.sync_copy(ar_hbm.at[pl.ds(ib, NPAD)], far)

    lanes = lax.iota(jnp.int32, LANES)
    lane0 = lanes == 0
    neg1 = jnp.full((LANES,), -1.0, jnp.float32)

    def _rot(v, sh):
        idx = (lanes + sh) & (LANES - 1)
        return lax.gather(
            v, idx[:, None],
            dimension_numbers=lax.GatherDimensionNumbers(
                offset_dims=(), collapsed_slice_dims=(0,),
                start_index_map=(0,)),
            slice_sizes=(1,),
            mode=lax.GatherScatterMode.PROMISE_IN_BOUNDS)

    def _lane_argmax(m, ix):
        for sh in (1, 2, 4, 8):
            m2 = _rot(m, sh)
            ix2 = _rot(ix, sh)
            better = (m2 > m) | ((m2 == m) & (ix2 < ix))
            m = jnp.where(better, m2, m)
            ix = jnp.where(better, ix2, ix)
        return m, ix

    m0 = s_ref[pl.ds(0, LANES)]
    ix0 = lanes + base
    for j in range(1, CHUNK // LANES):
        v = s_ref[pl.ds(j * LANES, LANES)]
        take = v > m0
        m0 = jnp.where(take, v, m0)
        ix0 = jnp.where(take, lanes + (base + j * LANES), ix0)
    m0, ix0 = _lane_argmax(m0, ix0)

    def step(k, carry):
        m, ix = carry

        p = k & 1
        stg[pl.ds(0, LANES)] = m
        stg[pl.ds(LANES, LANES)] = plsc.bitcast(ix, jnp.float32)
        pltpu.sync_copy(stg, shx.at[p, sid, pl.ds(0, 2 * LANES)])
        plsc.subcore_barrier()
        pltpu.sync_copy(shx.at[p], allv)
        wm = allv[0, pl.ds(0, LANES)]
        wi = plsc.bitcast(allv[0, pl.ds(LANES, LANES)], jnp.int32)
        for t in range(1, NSUB):
            vm = allv[t, pl.ds(0, LANES)]
            vi = plsc.bitcast(allv[t, pl.ds(LANES, LANES)], jnp.int32)
            better = (vm > wm) | ((vm == wm) & (vi < wi))
            wm = jnp.where(better, vm, wm)
            wi = jnp.where(better, vi, wi)

        wx = plsc.load_gather(fx, [wi])
        wy = plsc.load_gather(fy, [wi])
        ww = plsc.load_gather(fw, [wi])
        wh = plsc.load_gather(fh, [wi])
        wx1 = plsc.load_gather(fx1, [wi])
        wy1 = plsc.load_gather(fy1, [wi])
        wa = plsc.load_gather(far, [wi])

        ok = wm > 0.0
        kidx = jnp.full((LANES,), k, jnp.int32)
        zero = jnp.zeros((LANES,), jnp.float32)
        plsc.store_scatter(ks, [kidx], jnp.where(ok, wm, zero), mask=lane0)
        plsc.store_scatter(kx, [kidx], jnp.where(ok, wx, zero), mask=lane0)
        plsc.store_scatter(ky, [kidx], jnp.where(ok, wy, zero), mask=lane0)
        plsc.store_scatter(kw, [kidx], jnp.where(ok, ww, zero), mask=lane0)
        plsc.store_scatter(kh, [kidx], jnp.where(ok, wh, zero), mask=lane0)

        nm = neg1
        nix = lanes + base
        for j in range(CHUNK // LANES):
            sl = pl.ds(j * LANES, LANES)
            x0 = ox0[sl]
            y0 = oy0[sl]
            x1 = ox1[sl]
            y1 = oy1[sl]
            iw = jnp.maximum(jnp.minimum(wx1, x1) - jnp.maximum(wx, x0) + 1.0, 0.0)
            ih = jnp.maximum(jnp.minimum(wy1, y1) - jnp.maximum(wy, y0) + 1.0, 0.0)
            inter = iw * ih
            iou = inter / (wa + oar[sl] - inter + 1e-9)
            s = jnp.where(iou >= IOU_THRESH, neg1, s_ref[sl])
            s_ref[sl] = s
            take = s > nm
            nm = jnp.where(take, s, nm)
            nix = jnp.where(take, lanes + (base + j * LANES), nix)
        return _lane_argmax(nm, nix)

    lax.fori_loop(0, NMS_KEEP, step, (m0, ix0))

    @pl.when(sid == 0)
    def _():
        for j in range(KPAD // LANES):
            sl = pl.ds(j * LANES, LANES)
            kx1 = kx[sl] + kw[sl] - 1.0
            ky1 = ky[sl] + kh[sl] - 1.0
            kw[sl] = kx1
            kh[sl] = ky1
        ob = img * (4 * KPAD)
        pltpu.sync_copy(ks, out_s_hbm.at[pl.ds(img * KPAD, KPAD)])
        pltpu.sync_copy(kx, out_b_hbm.at[pl.ds(ob + 0 * KPAD, KPAD)])
        pltpu.sync_copy(ky, out_b_hbm.at[pl.ds(ob + 1 * KPAD, KPAD)])
        pltpu.sync_copy(kw, out_b_hbm.at[pl.ds(ob + 2 * KPAD, KPAD)])
        pltpu.sync_copy(kh, out_b_hbm.at[pl.ds(ob + 3 * KPAD, KPAD)])


@functools.cache
def _make_nms_sc():
    return functools.partial(
        pl.kernel,
        out_type=[
            jax.ShapeDtypeStruct((B * KPAD,), jnp.float32),
            jax.ShapeDtypeStruct((B * 4 * KPAD,), jnp.float32),
        ],
        mesh=plsc.VectorSubcoreMesh(core_axis_name="c", subcore_axis_name="s"),
        compiler_params=pltpu.CompilerParams(needs_layout_passes=False),
        scratch_types=[
        pltpu.VMEM((CHUNK,), jnp.float32),
        pltpu.VMEM((CHUNK,), jnp.float32),
        pltpu.VMEM((CHUNK,), jnp.float32),
        pltpu.VMEM((CHUNK,), jnp.float32),
        pltpu.VMEM((CHUNK,), jnp.float32),
        pltpu.VMEM((CHUNK,), jnp.float32),
        pltpu.VMEM((NPAD,), jnp.float32),
        pltpu.VMEM((NPAD,), jnp.float32),
        pltpu.VMEM((NPAD,), jnp.float32),
        pltpu.VMEM((NPAD,), jnp.float32),
        pltpu.VMEM((NPAD,), jnp.float32),
        pltpu.VMEM((NPAD,), jnp.float32),
        pltpu.VMEM((NPAD,), jnp.float32),
        pltpu.VMEM((KPAD,), jnp.float32),
        pltpu.VMEM((KPAD,), jnp.float32),
        pltpu.VMEM((KPAD,), jnp.float32),
        pltpu.VMEM((KPAD,), jnp.float32),
        pltpu.VMEM((KPAD,), jnp.float32),
        pltpu.VMEM((2 * LANES,), jnp.float32),
        pltpu.VMEM((NSUB, 128), jnp.float32),
            pltpu.VMEM_SHARED((2, NSUB, 128), jnp.float32),
        ],
    )(_nms_body)


def kernel(rois, proposals, W1, b1, Wc, bc, Wr, br):
    rois = rois.reshape(B, N, D)
    pad = NPAD - N
    rois_p = jnp.pad(rois, ((0, 0), (0, pad), (0, 0)))
    prop_p = jnp.pad(proposals, ((0, 0), (0, pad), (0, 0)))
    pcols = [prop_p[:, :, j].reshape(-1) for j in range(4)]
    rowid = jnp.arange(NPAD, dtype=jnp.int32)[None, :]
    Wcr = jnp.concatenate([Wc, Wr], axis=1)
    bcr = jnp.concatenate([bc, br], axis=0)[None, :]
    packed = _stage1(rois_p, *pcols, rowid, W1, b1[None, :], Wcr, bcr)
    out_s, out_b = _make_nms_sc()(*packed)
    clss_out = out_s.reshape(B, KPAD)[:, :NMS_KEEP]
    out_b = out_b.reshape(B, 4, KPAD)
    bboxes_out = jnp.transpose(out_b, (0, 2, 1))[:, :NMS_KEEP, :]
    return (clss_out, bboxes_out)


# speedup vs baseline: 1.1649x; 1.0203x over previous
import functools

import jax
import jax.numpy as jnp
from jax import lax
from jax.experimental import pallas as pl
from jax.experimental.pallas import tpu as pltpu
from jax.experimental.pallas import tpu_sc as plsc

IMG_SIZE = 512
B, N, D = 2, 5000, 256
HID, NCLS = 4096, 81
NMS_KEEP = 100
IOU_THRESH = 0.5

NPAD = 5120
ROW_T = 1024
NSUB = 16
LANES = 16
CHUNK = NPAD // NSUB
KPAD = 128


def _stage1_body(rois_ref, p0_ref, p1_ref, p2_ref, p3_ref, rowid_ref,
                 w1_ref, b1_ref, wcr_ref, bcr_ref,
                 s_o, x_o, y_o, w_o, h_o, x1_o, y1_o, ar_o):
    x = jnp.dot(rois_ref[0], w1_ref[...],
                preferred_element_type=jnp.float32) + b1_ref[...]
    y = jnp.dot(x, wcr_ref[...], preferred_element_type=jnp.float32) + bcr_ref[...]
    clss = y[:, :NCLS]
    m2 = jnp.max(clss, axis=1, keepdims=True)
    ssum2 = jnp.sum(jnp.exp(clss - m2), axis=1, keepdims=True)
    z = jnp.concatenate(
        [m2, ssum2, clss[:, 0:1], y[:, NCLS:NCLS + 1], y[:, NCLS + 1:NCLS + 2],
         y[:, NCLS + 2:NCLS + 3], y[:, NCLS + 3:NCLS + 4], m2], axis=1)
    zT = z.T
    m = zT[0, :]
    ssum = zT[1, :]
    c0 = zT[2, :]
    reg0 = zT[3, :]
    reg1 = zT[4, :]
    reg2 = zT[5, :]
    reg3 = zT[6, :]
    score = 1.0 / ssum
    valid = (c0 < m) & (score >= 0.01) & (rowid_ref[0] < N)
    score = jnp.where(valid, score, 0.0)

    p0 = p0_ref[...]
    p1 = p1_ref[...]
    p2 = p2_ref[...]
    p3 = p3_ref[...]
    px = p0 + p2 * reg0
    py = p1 + p3 * reg1
    pw = p2 * jnp.exp(reg2)
    ph = p3 * jnp.exp(reg3)
    bx = jnp.clip(px, 0.0, IMG_SIZE - 1.0)
    by = jnp.clip(py, 0.0, IMG_SIZE - 1.0)
    bw = jnp.clip(pw, 1.0, float(IMG_SIZE))
    bh = jnp.clip(ph, 1.0, float(IMG_SIZE))
    x1 = bx + bw - 1.0
    y1 = by + bh - 1.0
    area = (x1 - bx + 1.0) * (y1 - by + 1.0)

    s_o[...] = score
    x_o[...] = bx
    y_o[...] = by
    w_o[...] = bw
    h_o[...] = bh
    x1_o[...] = x1
    y1_o[...] = y1
    ar_o[...] = area


def _stage1(rois_p, p0, p1, p2, p3, rowid, W1, b1r, Wcr, bcr):
    nt = NPAD // ROW_T
    flat = pl.BlockSpec((ROW_T,), lambda b, t: (b * nt + t,))
    return pl.pallas_call(
        _stage1_body,
        grid=(B, nt),
        in_specs=[
            pl.BlockSpec((1, ROW_T, D), lambda b, t: (b, t, 0)),
            flat, flat, flat, flat,
            pl.BlockSpec((1, ROW_T), lambda b, t: (0, t)),
            pl.BlockSpec((D, HID), lambda b, t: (0, 0)),
            pl.BlockSpec((1, HID), lambda b, t: (0, 0)),
            pl.BlockSpec((HID, NCLS + 4), lambda b, t: (0, 0)),
            pl.BlockSpec((1, NCLS + 4), lambda b, t: (0, 0)),
        ],
        out_specs=[flat] * 8,
        out_shape=[jax.ShapeDtypeStruct((B * NPAD,), jnp.float32)] * 8,
    )(rois_p, p0, p1, p2, p3, rowid, W1, b1r, Wcr, bcr)


def _nms_body(s_hbm, x_hbm, y_hbm, w_hbm, h_hbm, x1_hbm, y1_hbm, ar_hbm,
              out_s_hbm, out_b_hbm,
              s_ref, ox0, oy0, ox1, oy1, oar,
              fx, fy, fw, fh,
              ks, kx, ky, kw, kh,
              stg, allv, shx):
    img = lax.axis_index("c")
    sid = lax.axis_index("s")
    base = sid * CHUNK
    ib = img * NPAD

    pltpu.sync_copy(s_hbm.at[pl.ds(ib + base, CHUNK)], s_ref)
    pltpu.sync_copy(x_hbm.at[pl.ds(ib + base, CHUNK)], ox0)
    pltpu.sync_copy(y_hbm.at[pl.ds(ib + base, CHUNK)], oy0)
    pltpu.sync_copy(x1_hbm.at[pl.ds(ib + base, CHUNK)], ox1)
    pltpu.sync_copy(y1_hbm.at[pl.ds(ib + base, CHUNK)], oy1)
    pltpu.sync_copy(ar_hbm.at[pl.ds(ib + base, CHUNK)], oar)
    pltpu.sync_copy(x_hbm.at[pl.ds(ib, NPAD)], fx)
    pltpu.sync_copy(y_hbm.at[pl.ds(ib, NPAD)], fy)
    pltpu.sync_copy(w_hbm.at[pl.ds(ib, NPAD)], fw)
    pltpu.sync_copy(h_hbm.at[pl.ds(ib, NPAD)], fh)

    lanes = lax.iota(jnp.int32, LANES)
    lane0 = lanes == 0
    neg1 = jnp.full((LANES,), -1.0, jnp.float32)

    def _rot(v, sh):
        idx = (lanes + sh) & (LANES - 1)
        return lax.gather(
            v, idx[:, None],
            dimension_numbers=lax.GatherDimensionNumbers(
                offset_dims=(), collapsed_slice_dims=(0,),
                start_index_map=(0,)),
            slice_sizes=(1,),
            mode=lax.GatherScatterMode.PROMISE_IN_BOUNDS)

    def _lane_argmax(m, ix):
        for sh in (1, 2, 4, 8):
            m2 = _rot(m, sh)
            ix2 = _rot(ix, sh)
            better = (m2 > m) | ((m2 == m) & (ix2 < ix))
            m = jnp.where(better, m2, m)
            ix = jnp.where(better, ix2, ix)
        return m, ix

    m0 = s_ref[pl.ds(0, LANES)]
    ix0 = lanes + base
    for j in range(1, CHUNK // LANES):
        v = s_ref[pl.ds(j * LANES, LANES)]
        take = v > m0
        m0 = jnp.where(take, v, m0)
        ix0 = jnp.where(take, lanes + (base + j * LANES), ix0)
    m0, ix0 = _lane_argmax(m0, ix0)

    def step(k, carry):
        m, ix = carry

        p = k & 1
        stg[pl.ds(0, LANES)] = m
        stg[pl.ds(LANES, LANES)] = plsc.bitcast(ix, jnp.float32)
        pltpu.sync_copy(stg, shx.at[p, sid, pl.ds(0, 2 * LANES)])
        plsc.subcore_barrier()
        pltpu.sync_copy(shx.at[p], allv)
        wm = allv[0, pl.ds(0, LANES)]
        wi = plsc.bitcast(allv[0, pl.ds(LANES, LANES)], jnp.int32)
        for t in range(1, NSUB):
            vm = allv[t, pl.ds(0, LANES)]
            vi = plsc.bitcast(allv[t, pl.ds(LANES, LANES)], jnp.int32)
            better = (vm > wm) | ((vm == wm) & (vi < wi))
            wm = jnp.where(better, vm, wm)
            wi = jnp.where(better, vi, wi)

        wx = plsc.load_gather(fx, [wi])
        wy = plsc.load_gather(fy, [wi])
        ww = plsc.load_gather(fw, [wi])
        wh = plsc.load_gather(fh, [wi])
        wx1 = wx + ww - 1.0
        wy1 = wy + wh - 1.0
        wa = (wx1 - wx + 1.0) * (wy1 - wy + 1.0)

        ok = wm > 0.0
        kidx = jnp.full((LANES,), k, jnp.int32)
        zero = jnp.zeros((LANES,), jnp.float32)
        plsc.store_scatter(ks, [kidx], jnp.where(ok, wm, zero), mask=lane0)
        plsc.store_scatter(kx, [kidx], jnp.where(ok, wx, zero), mask=lane0)
        plsc.store_scatter(ky, [kidx], jnp.where(ok, wy, zero), mask=lane0)
        plsc.store_scatter(kw, [kidx], jnp.where(ok, ww, zero), mask=lane0)
        plsc.store_scatter(kh, [kidx], jnp.where(ok, wh, zero), mask=lane0)

        nm = neg1
        nix = lanes + base
        for j in range(CHUNK // LANES):
            sl = pl.ds(j * LANES, LANES)
            x0 = ox0[sl]
            y0 = oy0[sl]
            x1 = ox1[sl]
            y1 = oy1[sl]
            iw = jnp.maximum(jnp.minimum(wx1, x1) - jnp.maximum(wx, x0) + 1.0, 0.0)
            ih = jnp.maximum(jnp.minimum(wy1, y1) - jnp.maximum(wy, y0) + 1.0, 0.0)
            inter = iw * ih
            iou = inter / (wa + oar[sl] - inter + 1e-9)
            s = jnp.where(iou >= IOU_THRESH, neg1, s_ref[sl])
            s_ref[sl] = s
            take = s > nm
            nm = jnp.where(take, s, nm)
            nix = jnp.where(take, lanes + (base + j * LANES), nix)
        return _lane_argmax(nm, nix)

    lax.fori_loop(0, NMS_KEEP, step, (m0, ix0))

    @pl.when(sid == 0)
    def _():
        for j in range(KPAD // LANES):
            sl = pl.ds(j * LANES, LANES)
            kx1 = kx[sl] + kw[sl] - 1.0
            ky1 = ky[sl] + kh[sl] - 1.0
            kw[sl] = kx1
            kh[sl] = ky1
        ob = img * (4 * KPAD)
        pltpu.sync_copy(ks, out_s_hbm.at[pl.ds(img * KPAD, KPAD)])
        pltpu.sync_copy(kx, out_b_hbm.at[pl.ds(ob + 0 * KPAD, KPAD)])
        pltpu.sync_copy(ky, out_b_hbm.at[pl.ds(ob + 1 * KPAD, KPAD)])
        pltpu.sync_copy(kw, out_b_hbm.at[pl.ds(ob + 2 * KPAD, KPAD)])
        pltpu.sync_copy(kh, out_b_hbm.at[pl.ds(ob + 3 * KPAD, KPAD)])


@functools.cache
def _make_nms_sc():
    return functools.partial(
        pl.kernel,
        out_type=[
            jax.ShapeDtypeStruct((B * KPAD,), jnp.float32),
            jax.ShapeDtypeStruct((B * 4 * KPAD,), jnp.float32),
        ],
        mesh=plsc.VectorSubcoreMesh(core_axis_name="c", subcore_axis_name="s"),
        compiler_params=pltpu.CompilerParams(needs_layout_passes=False),
        scratch_types=[
        pltpu.VMEM((CHUNK,), jnp.float32),
        pltpu.VMEM((CHUNK,), jnp.float32),
        pltpu.VMEM((CHUNK,), jnp.float32),
        pltpu.VMEM((CHUNK,), jnp.float32),
        pltpu.VMEM((CHUNK,), jnp.float32),
        pltpu.VMEM((CHUNK,), jnp.float32),
        pltpu.VMEM((NPAD,), jnp.float32),
        pltpu.VMEM((NPAD,), jnp.float32),
        pltpu.VMEM((NPAD,), jnp.float32),
        pltpu.VMEM((NPAD,), jnp.float32),
        pltpu.VMEM((KPAD,), jnp.float32),
        pltpu.VMEM((KPAD,), jnp.float32),
        pltpu.VMEM((KPAD,), jnp.float32),
        pltpu.VMEM((KPAD,), jnp.float32),
        pltpu.VMEM((KPAD,), jnp.float32),
        pltpu.VMEM((2 * LANES,), jnp.float32),
        pltpu.VMEM((NSUB, 128), jnp.float32),
            pltpu.VMEM_SHARED((2, NSUB, 128), jnp.float32),
        ],
    )(_nms_body)


def kernel(rois, proposals, W1, b1, Wc, bc, Wr, br):
    rois = rois.reshape(B, N, D)
    pad = NPAD - N
    rois_p = jnp.pad(rois, ((0, 0), (0, pad), (0, 0)))
    prop_p = jnp.pad(proposals, ((0, 0), (0, pad), (0, 0)))
    pcols = [prop_p[:, :, j].reshape(-1) for j in range(4)]
    rowid = jnp.arange(NPAD, dtype=jnp.int32)[None, :]
    Wcr = jnp.concatenate([Wc, Wr], axis=1)
    bcr = jnp.concatenate([bc, br], axis=0)[None, :]
    packed = _stage1(rois_p, *pcols, rowid, W1, b1[None, :], Wcr, bcr)
    out_s, out_b = _make_nms_sc()(*packed)
    clss_out = out_s.reshape(B, KPAD)[:, :NMS_KEEP]
    out_b = out_b.reshape(B, 4, KPAD)
    bboxes_out = jnp.transpose(out_b, (0, 2, 1))[:, :NMS_KEEP, :]
    return (clss_out, bboxes_out)


# speedup vs baseline: 1.2337x; 1.0590x over previous
import functools

import jax
import jax.numpy as jnp
from jax import lax
from jax.experimental import pallas as pl
from jax.experimental.pallas import tpu as pltpu
from jax.experimental.pallas import tpu_sc as plsc

IMG_SIZE = 512
B, N, D = 2, 5000, 256
HID, NCLS = 4096, 81
NMS_KEEP = 100
IOU_THRESH = 0.5

NPAD = 5120
ROW_T = 1024
NSUB = 16
LANES = 16
CHUNK = NPAD // NSUB
KPAD = 128


def _stage1_body(rois_ref, p0_ref, p1_ref, p2_ref, p3_ref, rowid_ref,
                 w1_ref, b1_ref, wcr_ref, bcr_ref,
                 s_o, x_o, y_o, w_o, h_o, x1_o, y1_o, ar_o):
    x = jnp.dot(rois_ref[0], w1_ref[...],
                preferred_element_type=jnp.float32) + b1_ref[...]
    y = jnp.dot(x, wcr_ref[...], preferred_element_type=jnp.float32) + bcr_ref[...]
    clss = y[:, :NCLS]
    m2 = jnp.max(clss, axis=1, keepdims=True)
    ssum2 = jnp.sum(jnp.exp(clss - m2), axis=1, keepdims=True)
    z = jnp.concatenate(
        [m2, ssum2, clss[:, 0:1], y[:, NCLS:NCLS + 1], y[:, NCLS + 1:NCLS + 2],
         y[:, NCLS + 2:NCLS + 3], y[:, NCLS + 3:NCLS + 4], m2], axis=1)
    zT = z.T
    m = zT[0, :]
    ssum = zT[1, :]
    c0 = zT[2, :]
    reg0 = zT[3, :]
    reg1 = zT[4, :]
    reg2 = zT[5, :]
    reg3 = zT[6, :]
    score = 1.0 / ssum
    valid = (c0 < m) & (score >= 0.01) & (rowid_ref[0] < N)
    score = jnp.where(valid, score, 0.0)

    p0 = p0_ref[0, 0, :]
    p1 = p1_ref[0, 0, :]
    p2 = p2_ref[0, 0, :]
    p3 = p3_ref[0, 0, :]
    px = p0 + p2 * reg0
    py = p1 + p3 * reg1
    pw = p2 * jnp.exp(reg2)
    ph = p3 * jnp.exp(reg3)
    bx = jnp.clip(px, 0.0, IMG_SIZE - 1.0)
    by = jnp.clip(py, 0.0, IMG_SIZE - 1.0)
    bw = jnp.clip(pw, 1.0, float(IMG_SIZE))
    bh = jnp.clip(ph, 1.0, float(IMG_SIZE))
    x1 = bx + bw - 1.0
    y1 = by + bh - 1.0
    area = (x1 - bx + 1.0) * (y1 - by + 1.0)

    s_o[...] = score
    x_o[...] = bx
    y_o[...] = by
    w_o[...] = bw
    h_o[...] = bh
    x1_o[...] = x1
    y1_o[...] = y1
    ar_o[...] = area


def _stage1(rois_p, p0, p1, p2, p3, rowid, W1, b1r, Wcr, bcr):
    nt = NPAD // ROW_T
    flat = pl.BlockSpec((ROW_T,), lambda b, t: (b * nt + t,))
    return pl.pallas_call(
        _stage1_body,
        grid=(B, nt),
        in_specs=[
            pl.BlockSpec((1, ROW_T, D), lambda b, t: (b, t, 0)),
            pl.BlockSpec((1, 1, ROW_T), lambda b, t: (b, 0, t)),
            pl.BlockSpec((1, 1, ROW_T), lambda b, t: (b, 0, t)),
            pl.BlockSpec((1, 1, ROW_T), lambda b, t: (b, 0, t)),
            pl.BlockSpec((1, 1, ROW_T), lambda b, t: (b, 0, t)),
            pl.BlockSpec((1, ROW_T), lambda b, t: (0, t)),
            pl.BlockSpec((D, HID), lambda b, t: (0, 0)),
            pl.BlockSpec((1, HID), lambda b, t: (0, 0)),
            pl.BlockSpec((HID, NCLS + 4), lambda b, t: (0, 0)),
            pl.BlockSpec((1, NCLS + 4), lambda b, t: (0, 0)),
        ],
        out_specs=[flat] * 8,
        out_shape=[jax.ShapeDtypeStruct((B * NPAD,), jnp.float32)] * 8,
    )(rois_p, p0, p1, p2, p3, rowid, W1, b1r, Wcr, bcr)


def _nms_body(s_hbm, x_hbm, y_hbm, w_hbm, h_hbm, x1_hbm, y1_hbm, ar_hbm,
              out_s_hbm, out_b_hbm,
              s_ref, ox0, oy0, ox1, oy1, oar,
              fx, fy, fw, fh,
              ks, kx, ky, kw, kh,
              stg, allv, shx):
    img = lax.axis_index("c")
    sid = lax.axis_index("s")
    base = sid * CHUNK
    ib = img * NPAD

    pltpu.sync_copy(s_hbm.at[pl.ds(ib + base, CHUNK)], s_ref)
    pltpu.sync_copy(x_hbm.at[pl.ds(ib + base, CHUNK)], ox0)
    pltpu.sync_copy(y_hbm.at[pl.ds(ib + base, CHUNK)], oy0)
    pltpu.sync_copy(x1_hbm.at[pl.ds(ib + base, CHUNK)], ox1)
    pltpu.sync_copy(y1_hbm.at[pl.ds(ib + base, CHUNK)], oy1)
    pltpu.sync_copy(ar_hbm.at[pl.ds(ib + base, CHUNK)], oar)
    pltpu.sync_copy(x_hbm.at[pl.ds(ib, NPAD)], fx)
    pltpu.sync_copy(y_hbm.at[pl.ds(ib, NPAD)], fy)
    pltpu.sync_copy(w_hbm.at[pl.ds(ib, NPAD)], fw)
    pltpu.sync_copy(h_hbm.at[pl.ds(ib, NPAD)], fh)

    lanes = lax.iota(jnp.int32, LANES)
    lane0 = lanes == 0
    neg1 = jnp.full((LANES,), -1.0, jnp.float32)

    def _rot(v, sh):
        idx = (lanes + sh) & (LANES - 1)
        return lax.gather(
            v, idx[:, None],
            dimension_numbers=lax.GatherDimensionNumbers(
                offset_dims=(), collapsed_slice_dims=(0,),
                start_index_map=(0,)),
            slice_sizes=(1,),
            mode=lax.GatherScatterMode.PROMISE_IN_BOUNDS)

    def _lane_argmax(m, ix):
        for sh in (1, 2, 4, 8):
            m2 = _rot(m, sh)
            ix2 = _rot(ix, sh)
            better = (m2 > m) | ((m2 == m) & (ix2 < ix))
            m = jnp.where(better, m2, m)
            ix = jnp.where(better, ix2, ix)
        return m, ix

    m0 = s_ref[pl.ds(0, LANES)]
    ix0 = lanes + base
    for j in range(1, CHUNK // LANES):
        v = s_ref[pl.ds(j * LANES, LANES)]
        take = v > m0
        m0 = jnp.where(take, v, m0)
        ix0 = jnp.where(take, lanes + (base + j * LANES), ix0)
    m0, ix0 = _lane_argmax(m0, ix0)

    def step(k, carry):
        m, ix = carry

        p = k & 1
        stg[pl.ds(0, LANES)] = m
        stg[pl.ds(LANES, LANES)] = plsc.bitcast(ix, jnp.float32)
        pltpu.sync_copy(stg, shx.at[p, sid, pl.ds(0, 2 * LANES)])
        plsc.subcore_barrier()
        pltpu.sync_copy(shx.at[p], allv)
        wm = allv[0, pl.ds(0, LANES)]
        wi = plsc.bitcast(allv[0, pl.ds(LANES, LANES)], jnp.int32)
        for t in range(1, NSUB):
            vm = allv[t, pl.ds(0, LANES)]
            vi = plsc.bitcast(allv[t, pl.ds(LANES, LANES)], jnp.int32)
            better = (vm > wm) | ((vm == wm) & (vi < wi))
            wm = jnp.where(better, vm, wm)
            wi = jnp.where(better, vi, wi)

        wx = plsc.load_gather(fx, [wi])
        wy = plsc.load_gather(fy, [wi])
        ww = plsc.load_gather(fw, [wi])
        wh = plsc.load_gather(fh, [wi])
        wx1 = wx + ww - 1.0
        wy1 = wy + wh - 1.0
        wa = (wx1 - wx + 1.0) * (wy1 - wy + 1.0)

        ok = wm > 0.0
        kidx = jnp.full((LANES,), k, jnp.int32)
        zero = jnp.zeros((LANES,), jnp.float32)
        plsc.store_scatter(ks, [kidx], jnp.where(ok, wm, zero), mask=lane0)
        plsc.store_scatter(kx, [kidx], jnp.where(ok, wx, zero), mask=lane0)
        plsc.store_scatter(ky, [kidx], jnp.where(ok, wy, zero), mask=lane0)
        plsc.store_scatter(kw, [kidx], jnp.where(ok, ww, zero), mask=lane0)
        plsc.store_scatter(kh, [kidx], jnp.where(ok, wh, zero), mask=lane0)

        nm = neg1
        nix = lanes + base
        for j in range(CHUNK // LANES):
            sl = pl.ds(j * LANES, LANES)
            x0 = ox0[sl]
            y0 = oy0[sl]
            x1 = ox1[sl]
            y1 = oy1[sl]
            iw = jnp.maximum(jnp.minimum(wx1, x1) - jnp.maximum(wx, x0) + 1.0, 0.0)
            ih = jnp.maximum(jnp.minimum(wy1, y1) - jnp.maximum(wy, y0) + 1.0, 0.0)
            inter = iw * ih
            iou = inter / (wa + oar[sl] - inter + 1e-9)
            s = jnp.where(iou >= IOU_THRESH, neg1, s_ref[sl])
            s_ref[sl] = s
            take = s > nm
            nm = jnp.where(take, s, nm)
            nix = jnp.where(take, lanes + (base + j * LANES), nix)
        return _lane_argmax(nm, nix)

    lax.fori_loop(0, NMS_KEEP, step, (m0, ix0))

    @pl.when(sid == 0)
    def _():
        for j in range(KPAD // LANES):
            sl = pl.ds(j * LANES, LANES)
            kx1 = kx[sl] + kw[sl] - 1.0
            ky1 = ky[sl] + kh[sl] - 1.0
            kw[sl] = kx1
            kh[sl] = ky1
        ob = img * (4 * KPAD)
        pltpu.sync_copy(ks, out_s_hbm.at[pl.ds(img * KPAD, KPAD)])
        pltpu.sync_copy(kx, out_b_hbm.at[pl.ds(ob + 0 * KPAD, KPAD)])
        pltpu.sync_copy(ky, out_b_hbm.at[pl.ds(ob + 1 * KPAD, KPAD)])
        pltpu.sync_copy(kw, out_b_hbm.at[pl.ds(ob + 2 * KPAD, KPAD)])
        pltpu.sync_copy(kh, out_b_hbm.at[pl.ds(ob + 3 * KPAD, KPAD)])


@functools.cache
def _make_nms_sc():
    return functools.partial(
        pl.kernel,
        out_type=[
            jax.ShapeDtypeStruct((B * KPAD,), jnp.float32),
            jax.ShapeDtypeStruct((B * 4 * KPAD,), jnp.float32),
        ],
        mesh=plsc.VectorSubcoreMesh(core_axis_name="c", subcore_axis_name="s"),
        compiler_params=pltpu.CompilerParams(needs_layout_passes=False),
        scratch_types=[
        pltpu.VMEM((CHUNK,), jnp.float32),
        pltpu.VMEM((CHUNK,), jnp.float32),
        pltpu.VMEM((CHUNK,), jnp.float32),
        pltpu.VMEM((CHUNK,), jnp.float32),
        pltpu.VMEM((CHUNK,), jnp.float32),
        pltpu.VMEM((CHUNK,), jnp.float32),
        pltpu.VMEM((NPAD,), jnp.float32),
        pltpu.VMEM((NPAD,), jnp.float32),
        pltpu.VMEM((NPAD,), jnp.float32),
        pltpu.VMEM((NPAD,), jnp.float32),
        pltpu.VMEM((KPAD,), jnp.float32),
        pltpu.VMEM((KPAD,), jnp.float32),
        pltpu.VMEM((KPAD,), jnp.float32),
        pltpu.VMEM((KPAD,), jnp.float32),
        pltpu.VMEM((KPAD,), jnp.float32),
        pltpu.VMEM((2 * LANES,), jnp.float32),
        pltpu.VMEM((NSUB, 128), jnp.float32),
            pltpu.VMEM_SHARED((2, NSUB, 128), jnp.float32),
        ],
    )(_nms_body)


def kernel(rois, proposals, W1, b1, Wc, bc, Wr, br):
    rois = rois.reshape(B, N, D)
    pcols = [proposals[:, :, j][:, None, :] for j in range(4)]
    rowid = jnp.arange(NPAD, dtype=jnp.int32)[None, :]
    Wcr = jnp.concatenate([Wc, Wr], axis=1)
    bcr = jnp.concatenate([bc, br], axis=0)[None, :]
    packed = _stage1(rois, *pcols, rowid, W1, b1[None, :], Wcr, bcr)
    out_s, out_b = _make_nms_sc()(*packed)
    clss_out = out_s.reshape(B, KPAD)[:, :NMS_KEEP]
    out_b = out_b.reshape(B, 4, KPAD)
    bboxes_out = jnp.transpose(out_b, (0, 2, 1))[:, :NMS_KEEP, :]
    return (clss_out, bboxes_out)
